# Initial kernel scaffold; baseline (speedup 1.0000x reference)
import jax
import jax.numpy as jnp
from jax.experimental import pallas as pl


def kernel(seq_embed, go_embed, adj, mlp_w1, mlp_b1, mlp_w2, mlp_b2, gc1_w, gc1_b, gc2_w, gc2_b, fc1_w, fc1_b, fc2_w, fc2_b):
    raise NotImplementedError("write your pallas kernel here")


# speedup vs baseline: 1.0115x; 1.0115x over previous
import functools

import jax
import jax.numpy as jnp
from jax.experimental import pallas as pl
from jax.experimental.pallas import tpu as pltpu


def _dot(a, b):
    return jax.lax.dot_general(
        a, b, (((1,), (0,)), ((), ())), preferred_element_type=jnp.float32
    )


def _dot_t(a, b):
    return jax.lax.dot_general(
        a, b, (((1,), (1,)), ((), ())), preferred_element_type=jnp.float32
    )


def _go_prep_body(go_ref, mw1_ref, mb1_ref, mw2_ref, mb2_ref, gw1_ref,
                  hsem_ref, s1_ref):
    g = go_ref[...]
    h = jnp.maximum(_dot(g, mw1_ref[...]) + mb1_ref[...], 0.0)
    hsem_ref[...] = _dot(h, mw2_ref[...]) + mb2_ref[...]
    s1_ref[...] = _dot(g, gw1_ref[...])


def _seq_prep_body(seq_ref, w1_ref, b1_ref, w2_ref, b2_ref, out_ref):
    h = jnp.maximum(_dot(seq_ref[...], w1_ref[...]) + b1_ref[...], 0.0)
    out_ref[...] = _dot(h, w2_ref[...]) + b2_ref[...]


def _pass1_body(adj_ref, s1_ref, b1_ref, gw2_ref, s2_ref):
    x = jnp.maximum(_dot(adj_ref[...], s1_ref[...]) + b1_ref[...], 0.0)
    s2_ref[...] = _dot(x, gw2_ref[...])


def _pass2_body(adj_ref, s2_ref, b2_ref, hsem_ref, seq_out_ref,
                hstruct_ref, pred_ref):
    hs = jnp.maximum(_dot(adj_ref[...], s2_ref[...]) + b2_ref[...], 0.0)
    hstruct_ref[...] = hs
    go_blk = jnp.concatenate([hsem_ref[...], hs], axis=1)
    pred_ref[...] = jax.nn.sigmoid(_dot_t(seq_out_ref[...], go_blk))


def _full(shape):
    return pl.BlockSpec(shape, lambda i: (0,) * len(shape))


def kernel(seq_embed, go_embed, adj, mlp_w1, mlp_b1, mlp_w2, mlp_b2,
           gc1_w, gc1_b, gc2_w, gc2_b, fc1_w, fc1_b, fc2_w, fc2_b):
    N, _ = adj.shape
    B, d_seq = seq_embed.shape
    go_feat = go_embed.shape[1]
    h0 = mlp_w1.shape[1]
    h1 = mlp_w2.shape[1]

    mb1 = mlp_b1.reshape(1, h0)
    mb2 = mlp_b2.reshape(1, h1)
    g1b = gc1_b.reshape(1, h0)
    g2b = gc2_b.reshape(1, h1)
    f1b = fc1_b.reshape(1, h0)
    f2b = fc2_b.reshape(1, 2 * h1)

    gr = min(1024, N)
    h_semantic, s1 = pl.pallas_call(
        _go_prep_body,
        grid=(N // gr,),
        in_specs=[
            pl.BlockSpec((gr, go_feat), lambda i: (i, 0)),
            _full((go_feat, h0)), _full((1, h0)),
            _full((h0, h1)), _full((1, h1)),
            _full((go_feat, h0)),
        ],
        out_specs=[
            pl.BlockSpec((gr, h1), lambda i: (i, 0)),
            pl.BlockSpec((gr, h0), lambda i: (i, 0)),
        ],
        out_shape=[
            jax.ShapeDtypeStruct((N, h1), jnp.float32),
            jax.ShapeDtypeStruct((N, h0), jnp.float32),
        ],
    )(go_embed, mlp_w1, mb1, mlp_w2, mb2, gc1_w)

    sr = min(1024, B)
    seq_out = pl.pallas_call(
        _seq_prep_body,
        grid=(B // sr,),
        in_specs=[
            pl.BlockSpec((sr, d_seq), lambda i: (i, 0)),
            _full((d_seq, h0)), _full((1, h0)),
            _full((h0, 2 * h1)), _full((1, 2 * h1)),
        ],
        out_specs=pl.BlockSpec((sr, 2 * h1), lambda i: (i, 0)),
        out_shape=jax.ShapeDtypeStruct((B, 2 * h1), jnp.float32),
    )(seq_embed, fc1_w, f1b, fc2_w, f2b)

    ar = min(256, N)
    s2 = pl.pallas_call(
        _pass1_body,
        grid=(N // ar,),
        in_specs=[
            pl.BlockSpec((ar, N), lambda i: (i, 0)),
            _full((N, h0)), _full((1, h0)), _full((h0, h1)),
        ],
        out_specs=pl.BlockSpec((ar, h1), lambda i: (i, 0)),
        out_shape=jax.ShapeDtypeStruct((N, h1), jnp.float32),
    )(adj, s1, g1b, gc2_w)

    h_structure, pred = pl.pallas_call(
        _pass2_body,
        grid=(N // ar,),
        in_specs=[
            pl.BlockSpec((ar, N), lambda i: (i, 0)),
            _full((N, h1)), _full((1, h1)),
            pl.BlockSpec((ar, h1), lambda i: (i, 0)),
            _full((B, 2 * h1)),
        ],
        out_specs=[
            pl.BlockSpec((ar, h1), lambda i: (i, 0)),
            pl.BlockSpec((B, ar), lambda i: (0, i)),
        ],
        out_shape=[
            jax.ShapeDtypeStruct((N, h1), jnp.float32),
            jax.ShapeDtypeStruct((B, N), jnp.float32),
        ],
    )(adj, s2, g2b, h_semantic, seq_out)

    return (h_semantic, h_structure, pred)


# speedup vs baseline: 1.0683x; 1.0561x over previous
import functools

import jax
import jax.numpy as jnp
from jax import lax
from jax.experimental import pallas as pl
from jax.experimental.pallas import tpu as pltpu


def _dot(a, b):
    return lax.dot_general(
        a, b, (((1,), (0,)), ((), ())), preferred_element_type=jnp.float32
    )


def _dot_t(a, b):
    return lax.dot_general(
        a, b, (((1,), (1,)), ((), ())), preferred_element_type=jnp.float32
    )


def _seq_prep_body(seq_ref, w1_ref, b1_ref, w2_ref, b2_ref, out_ref):
    h = jnp.maximum(_dot(seq_ref[...], w1_ref[...]) + b1_ref[...], 0.0)
    out_ref[...] = _dot(h, w2_ref[...]) + b2_ref[...]


def _full(shape):
    return pl.BlockSpec(shape, lambda i: (0,) * len(shape))


def _make_merged_body(nb, ar):
    def body(go_ref, adj_ref, mw1_ref, mb1_ref, mw2_ref, mb2_ref,
             gw1_ref, g1b_ref, gw2_ref, g2b_ref, seq_out_ref,
             hsem_ref, hstruct_ref, pred_ref, s1_ref, s2_ref):
        i = pl.program_id(0)

        @pl.when(i == 0)
        def _prologue():
            g = go_ref[...]
            h = jnp.maximum(_dot(g, mw1_ref[...]) + mb1_ref[...], 0.0)
            hsem_ref[...] = _dot(h, mw2_ref[...]) + mb2_ref[...]
            s1_ref[...] = _dot(g, gw1_ref[...])

        @pl.when(i < nb)
        def _phase1():
            x = jnp.maximum(_dot(adj_ref[...], s1_ref[...]) + g1b_ref[...],
                            0.0)
            s2_ref[pl.ds(i * ar, ar), :] = _dot(x, gw2_ref[...])

        @pl.when(i >= nb)
        def _phase2():
            j = i - nb
            hs = jnp.maximum(_dot(adj_ref[...], s2_ref[...]) + g2b_ref[...],
                             0.0)
            hstruct_ref[...] = hs
            hsem_blk = hsem_ref[pl.ds(j * ar, ar), :]
            go_blk = jnp.concatenate([hsem_blk, hs], axis=1)
            pred_ref[...] = jax.nn.sigmoid(_dot_t(seq_out_ref[...], go_blk))

    return body


def kernel(seq_embed, go_embed, adj, mlp_w1, mlp_b1, mlp_w2, mlp_b2,
           gc1_w, gc1_b, gc2_w, gc2_b, fc1_w, fc1_b, fc2_w, fc2_b):
    N, _ = adj.shape
    B, d_seq = seq_embed.shape
    go_feat = go_embed.shape[1]
    h0 = mlp_w1.shape[1]
    h1 = mlp_w2.shape[1]

    mb1 = mlp_b1.reshape(1, h0)
    mb2 = mlp_b2.reshape(1, h1)
    g1b = gc1_b.reshape(1, h0)
    g2b = gc2_b.reshape(1, h1)
    f1b = fc1_b.reshape(1, h0)
    f2b = fc2_b.reshape(1, 2 * h1)

    sr = min(1024, B)
    seq_out = pl.pallas_call(
        _seq_prep_body,
        grid=(B // sr,),
        in_specs=[
            pl.BlockSpec((sr, d_seq), lambda i: (i, 0)),
            _full((d_seq, h0)), _full((1, h0)),
            _full((h0, 2 * h1)), _full((1, 2 * h1)),
        ],
        out_specs=pl.BlockSpec((sr, 2 * h1), lambda i: (i, 0)),
        out_shape=jax.ShapeDtypeStruct((B, 2 * h1), jnp.float32),
    )(seq_embed, fc1_w, f1b, fc2_w, f2b)

    ar = min(256, N)
    nb = N // ar
    h_semantic, h_structure, pred = pl.pallas_call(
        _make_merged_body(nb, ar),
        grid=(2 * nb,),
        in_specs=[
            _full((N, go_feat)),
            pl.BlockSpec((ar, N), lambda i: (i % nb, 0)),
            _full((go_feat, h0)), _full((1, h0)),
            _full((h0, h1)), _full((1, h1)),
            _full((go_feat, h0)), _full((1, h0)),
            _full((h0, h1)), _full((1, h1)),
            _full((B, 2 * h1)),
        ],
        out_specs=[
            _full((N, h1)),
            pl.BlockSpec((ar, h1), lambda i: (lax.max(i - nb, 0), 0)),
            pl.BlockSpec((B, ar), lambda i: (0, lax.max(i - nb, 0))),
        ],
        out_shape=[
            jax.ShapeDtypeStruct((N, h1), jnp.float32),
            jax.ShapeDtypeStruct((N, h1), jnp.float32),
            jax.ShapeDtypeStruct((B, N), jnp.float32),
        ],
        scratch_shapes=[
            pltpu.VMEM((N, h0), jnp.float32),
            pltpu.VMEM((N, h1), jnp.float32),
        ],
    )(go_embed, adj, mlp_w1, mb1, mlp_w2, mb2, gc1_w, g1b, gc2_w, g2b,
      seq_out)

    return (h_semantic, h_structure, pred)
